# baseline (speedup 1.0000x reference)
import functools

import jax
import jax.numpy as jnp
from jax import lax
from jax.experimental import pallas as pl
from jax.experimental.pallas import tpu as pltpu
from jax.experimental.pallas import tpu_sc as plsc

_B = 16384
_D = 64
_L = 16

_info = plsc.get_sparse_core_info()
_NC = _info.num_cores
_NS = _info.num_subcores
_NW = _NC * _NS
_BW = _B // _NW
_IC = 128
_NIC = _BW // _IC


def _sc_body(user_hbm, item_hbm, meta_hbm, ut_hbm, it_hbm, cw_hbm, mw_hbm,
             bias_hbm, out_hbm,
             idx_u, idx_i, rows_u, rows_i, meta_v, cw_v, mw_v, bias_v,
             out_v, sem):
    wid = lax.axis_index("s") * _NC + lax.axis_index("c")

    pltpu.sync_copy(user_hbm.at[wid], idx_u)
    pltpu.sync_copy(item_hbm.at[wid], idx_i)
    pltpu.sync_copy(cw_hbm, cw_v)
    pltpu.sync_copy(mw_hbm, mw_v)
    pltpu.sync_copy(bias_hbm, bias_v)

    copies = [pltpu.async_copy(meta_hbm.at[wid], meta_v, sem)]
    for j in range(_NIC):
        copies.append(pltpu.async_copy(
            ut_hbm.at[idx_u.at[j]], rows_u.at[pl.ds(j * _IC, _IC)], sem))
        copies.append(pltpu.async_copy(
            it_hbm.at[idx_i.at[j]], rows_i.at[pl.ds(j * _IC, _IC)], sem))
    for c in copies:
        c.wait()

    cw_regs = [cw_v[pl.ds(c * _L, _L)] for c in range(_D // _L)]
    mw_regs = [mw_v[pl.ds(c * _L, _L)] for c in range(_D // _L)]
    bias_vec = bias_v[...]
    lane = lax.iota(jnp.int32, 16)

    def group(g, carry):
        res = jnp.zeros((_L,), jnp.float32)
        for r in range(_L):
            b = g * _L + r
            acc = None
            for c in range(_D // _L):
                u = rows_u[b, pl.ds(c * _L, _L)]
                it = rows_i[b, pl.ds(c * _L, _L)]
                m = meta_v[b, pl.ds(c * _L, _L)]
                t = u * it * cw_regs[c] + m * mw_regs[c]
                acc = t if acc is None else acc + t
            tot = jnp.sum(acc)
            res = jnp.where(lane == r, tot, res)
        out_v[pl.ds(g * _L, _L)] = res + bias_vec
        return carry

    lax.fori_loop(0, _BW // _L, group, 0)
    pltpu.sync_copy(out_v, out_hbm.at[pl.ds(wid * _BW, _BW)])


@jax.jit
def _run(user_i, item_i, meta, user_table, item_table, cw, mw, bias16):
    mesh = plsc.VectorSubcoreMesh(core_axis_name="c", subcore_axis_name="s")
    f = functools.partial(
        pl.kernel,
        mesh=mesh,
        compiler_params=pltpu.CompilerParams(
            needs_layout_passes=False, use_tc_tiling_on_sc=False),
        out_type=jax.ShapeDtypeStruct((_B,), jnp.float32),
        scratch_types=[
            pltpu.VMEM((_NIC, _IC), jnp.int32),
            pltpu.VMEM((_NIC, _IC), jnp.int32),
            pltpu.VMEM((_BW, _D), jnp.float32),
            pltpu.VMEM((_BW, _D), jnp.float32),
            pltpu.VMEM((_BW, _D), jnp.float32),
            pltpu.VMEM((_D,), jnp.float32),
            pltpu.VMEM((_D,), jnp.float32),
            pltpu.VMEM((_L,), jnp.float32),
            pltpu.VMEM((_BW,), jnp.float32),
            pltpu.SemaphoreType.DMA,
        ],
    )(_sc_body)
    return f(user_i, item_i, meta, user_table, item_table, cw, mw, bias16)


def kernel(user, item, item_metadata, user_table, item_table, comb_w, comb_b,
           meta_w, meta_b, global_bias):
    user_i = user.astype(jnp.int32).reshape(_NW, _NIC, _IC)
    item_i = item.astype(jnp.int32).reshape(_NW, _NIC, _IC)
    meta = item_metadata.reshape(_NW, _BW, _D)
    cw = comb_w.reshape(_D)
    mw = meta_w.reshape(_D)
    bias16 = jnp.broadcast_to(comb_b + meta_b + global_bias, (_L,)).astype(jnp.float32)
    return _run(user_i, item_i, meta, user_table, item_table, cw, mw, bias16)
